# baseline (speedup 1.0000x reference)
import functools

import jax
import jax.numpy as jnp
from jax.experimental import pallas as pl
from jax.experimental.pallas import tpu as pltpu

EPS = 1e-5
B = 64


def _head_body(pooled_ref, w1_ref, b1_ref, w2_ref, b2_ref, out_ref):
    h = jnp.maximum(jnp.dot(pooled_ref[...], w1_ref[...],
                            preferred_element_type=jnp.float32) + b1_ref[...], 0.0)
    out_ref[...] = jnp.dot(h, w2_ref[...], preferred_element_type=jnp.float32) + b2_ref[...]


def _head(pooled, w1, b1, w2, b2):
    ncls = w2.shape[1]
    return pl.pallas_call(
        _head_body,
        out_shape=jax.ShapeDtypeStruct((pooled.shape[0], ncls), jnp.float32),
    )(pooled, w1, b1[None, :], w2, b2[None, :])


def _adj_agg(P, Q, s, t, dst, n_dst, p):
    m = jax.nn.relu(P[s] + Q[t])
    E = s.shape[0]
    S = jax.ops.segment_sum(m, dst, num_segments=n_dst)
    c = jax.ops.segment_sum(jnp.ones((E,), jnp.float32), dst, num_segments=n_dst)
    mean = jnp.sum(S, axis=0) / E
    var = jnp.sum(m * m, axis=0) / E - mean * mean
    alpha = p["g"] * jax.lax.rsqrt(var + EPS)
    beta = p["bt"] - alpha * mean
    return alpha * S + beta * c[:, None]


def _bn(x, g, bt):
    mean = jnp.mean(x, axis=0)
    var = jnp.var(x, axis=0)
    return g * (x - mean) * jax.lax.rsqrt(var + EPS) + bt


def _upd(x, p):
    h = jax.nn.relu(x @ p["W1"] + p["b1"])
    h = jax.nn.relu(h @ p["W2"] + p["b2"])
    return _bn(h, p["g"], p["bt"])


def kernel(x0, x1, x2, up_index_0, up_attr_idx_0, up_index_1, up_attr_idx_1,
           down_index_1, down_attr_idx_1, down_index_2, down_attr_idx_2,
           batch0, batch1, batch2, params):
    xs = [x0, x1, x2]
    for lp in params["layers"]:
        x0_, x1_, x2_ = xs
        d = x0_.shape[1]
        Wu, bu = lp["up"]["W"], lp["up"]["b"]
        Wd, bd = lp["down"]["W"], lp["down"]["b"]
        agg0 = _adj_agg(x0_ @ Wu[:d] + bu, x1_ @ Wu[d:], up_index_0[0], up_attr_idx_0,
                        up_index_0[1], x0_.shape[0], lp["up"])
        agg1u = _adj_agg(x1_ @ Wu[:d] + bu, x2_ @ Wu[d:], up_index_1[0], up_attr_idx_1,
                         up_index_1[1], x1_.shape[0], lp["up"])
        agg1d = _adj_agg(x1_ @ Wd[:d] + bd, x0_ @ Wd[d:], down_index_1[0], down_attr_idx_1,
                         down_index_1[1], x1_.shape[0], lp["down"])
        agg2 = _adj_agg(x2_ @ Wd[:d] + bd, x1_ @ Wd[d:], down_index_2[0], down_attr_idx_2,
                        down_index_2[1], x2_.shape[0], lp["down"])
        xs = [_upd(x0_ + agg0, lp["upd"]), _upd(x1_ + agg1u + agg1d, lp["upd"]),
              _upd(x2_ + agg2, lp["upd"])]
    pooled = (jax.ops.segment_sum(xs[0], batch0, num_segments=B)
              + jax.ops.segment_sum(xs[1], batch1, num_segments=B)
              + jax.ops.segment_sum(xs[2], batch2, num_segments=B))
    return _head(pooled, params["lin1W"], params["lin1b"], params["lin2W"], params["lin2b"])


# speedup vs baseline: 2.6058x; 2.6058x over previous
import functools

import jax
import jax.numpy as jnp
from jax import lax
from jax.experimental import pallas as pl
from jax.experimental.pallas import tpu as pltpu
from jax.experimental.pallas import tpu_sc as plsc

EPS = 1e-5
B = 64
_NT = 16
_NC = 2
_K = 128


def _rup(x, m):
    return (x + m - 1) // m * m


def _sc_edge_call(Tsrc, Tattr, sidx2, tidx2, didx, h, Nd, do_counts):
    Ndp = _rup(Nd + 1, 128)
    Ep = didx.shape[0]
    Et = Ep // _NT
    nc = Et // _K
    rpt = Ndp // _NT
    nv = h // 16

    mesh = plsc.VectorSubcoreMesh(core_axis_name="c", subcore_axis_name="s")
    out_type = [jax.ShapeDtypeStruct((_NC * Ndp, h), jnp.float32),
                jax.ShapeDtypeStruct((_NC * _NT * h,), jnp.float32)]
    scratch = [
        pltpu.VMEM((_K, h), jnp.float32),
        pltpu.VMEM((_K, h), jnp.float32),
        pltpu.VMEM((_K,), jnp.int32),
        pltpu.VMEM((_K,), jnp.int32),
        pltpu.VMEM((_K,), jnp.int32),
        pltpu.VMEM((h,), jnp.float32),
        pltpu.VMEM_SHARED((Ndp, h), jnp.float32),
        pltpu.SemaphoreType.DMA,
        pltpu.SemaphoreType.DMA,
    ]
    if do_counts:
        out_type.append(jax.ShapeDtypeStruct((Ndp, 16), jnp.float32))
        scratch += [pltpu.VMEM((_K, 16), jnp.float32),
                    pltpu.VMEM_SHARED((Ndp, 16), jnp.float32)]

    def body(Ts, Ta, si, ti, di, *rest):
        if do_counts:
            S_out, SQ_out, C_out = rest[:3]
            bufA, bufB, sbuf, tbuf, dbuf, sqbuf, acc, sem1, sem2, ones, cacc = rest[3:]
        else:
            S_out, SQ_out = rest[:2]
            bufA, bufB, sbuf, tbuf, dbuf, sqbuf, acc, sem1, sem2 = rest[2:]
        c = lax.axis_index("c")
        s = lax.axis_index("s")

        def zrow(r, carry):
            for j in range(nv):
                bufA[r, pl.ds(j * 16, 16)] = jnp.zeros((16,), jnp.float32)
            return carry
        lax.fori_loop(0, _K, zrow, 0)
        row0 = s * rpt
        off = 0
        while off < rpt:
            seg = min(_K, rpt - off)
            pltpu.sync_copy(bufA.at[pl.ds(0, seg)], acc.at[pl.ds(row0 + off, seg)])
            off += seg
        if do_counts:
            def zcrow(r, carry):
                ones[r, :] = jnp.zeros((16,), jnp.float32)
                return carry
            lax.fori_loop(0, _K, zcrow, 0)

            @pl.when(c == 0)
            def _():
                o = 0
                while o < rpt:
                    seg = min(_K, rpt - o)
                    pltpu.sync_copy(ones.at[pl.ds(0, seg)], cacc.at[pl.ds(row0 + o, seg)])
                    o += seg

            def orow(r, carry):
                ones[r, :] = jnp.ones((16,), jnp.float32)
                return carry
            lax.fori_loop(0, _K, orow, 0)
        plsc.subcore_barrier()

        ebase = c * Ep + s * Et
        dbase = s * Et

        def chunk(i, acc_sq):
            eoff = i * _K
            pltpu.sync_copy(si.at[pl.ds(ebase + eoff, _K)], sbuf)
            pltpu.sync_copy(ti.at[pl.ds(ebase + eoff, _K)], tbuf)
            pltpu.sync_copy(di.at[pl.ds(dbase + eoff, _K)], dbuf)
            cpA = pltpu.async_copy(Ts.at[sbuf], bufA, sem1)
            cpB = pltpu.async_copy(Ta.at[tbuf], bufB, sem2)
            cpA.wait()
            cpB.wait()

            def row(k, asq):
                new = []
                for j in range(nv):
                    a = bufA[k, pl.ds(j * 16, 16)]
                    b = bufB[k, pl.ds(j * 16, 16)]
                    m = jnp.maximum(a + b, 0.0)
                    bufA[k, pl.ds(j * 16, 16)] = m
                    new.append(asq[j] + m * m)
                return tuple(new)
            acc_sq = lax.fori_loop(0, _K, row, acc_sq)
            pltpu.sync_copy(bufA, acc.at[dbuf], add=True)
            if do_counts:
                @pl.when(c == 0)
                def _():
                    pltpu.sync_copy(ones, cacc.at[dbuf], add=True)
            return acc_sq

        zero_sq = tuple(jnp.zeros((16,), jnp.float32) for _ in range(nv))
        acc_sq = lax.fori_loop(0, nc, chunk, zero_sq)
        plsc.subcore_barrier()

        pltpu.sync_copy(acc.at[pl.ds(row0, rpt)],
                        S_out.at[pl.ds(c * Ndp + row0, rpt)])
        for j in range(nv):
            sqbuf[pl.ds(j * 16, 16)] = acc_sq[j]
        wid = c * _NT + s
        pltpu.sync_copy(sqbuf, SQ_out.at[pl.ds(wid * h, h)])
        if do_counts:
            @pl.when(c == 0)
            def _():
                pltpu.sync_copy(cacc.at[pl.ds(row0, rpt)], C_out.at[pl.ds(row0, rpt)])

    fn = pl.kernel(body, out_type=out_type, mesh=mesh, scratch_types=scratch,
                   compiler_params=pltpu.CompilerParams(use_tc_tiling_on_sc=False))
    return fn(Tsrc, Tattr, sidx2, tidx2, didx)


def _table(P, h):
    z = jnp.zeros((1, h), jnp.float32)
    return jnp.concatenate([P[:, :h], z, P[:, h:], z], axis=0)


def _prep_edges(s, t, d, Ns, Na, Nd):
    E = s.shape[0]
    Ep = _rup(E, _NT * _K)
    pad = Ep - E
    s = jnp.concatenate([s.astype(jnp.int32), jnp.full((pad,), Ns, jnp.int32)])
    t = jnp.concatenate([t.astype(jnp.int32), jnp.full((pad,), Na, jnp.int32)])
    d = jnp.concatenate([d.astype(jnp.int32), jnp.full((pad,), Nd, jnp.int32)])
    sidx2 = jnp.concatenate([s, s + (Ns + 1)])
    tidx2 = jnp.concatenate([t, t + (Na + 1)])
    return sidx2, tidx2, d


def _adj_agg_sc(P, Q, prep, p, E_real, Nd, counts):
    d = P.shape[1]
    h = d // 2
    do_counts = counts is None
    outs = _sc_edge_call(_table(P, h), _table(Q, h), *prep, h=h, Nd=Nd,
                         do_counts=do_counts)
    S_out, SQ_out = outs[0], outs[1]
    Ndp = _rup(Nd + 1, 128)
    S = jnp.concatenate([S_out[:Nd], S_out[Ndp:Ndp + Nd]], axis=1)
    sq = SQ_out.reshape(2, _NT, h).sum(axis=1).reshape(-1)
    if do_counts:
        counts = outs[2][:Nd, 0]
    mean = jnp.sum(S, axis=0) / E_real
    var = sq / E_real - mean * mean
    alpha = p["g"] * lax.rsqrt(var + EPS)
    beta = p["bt"] - alpha * mean
    return alpha * S + beta * counts[:, None], counts


def _head_body(pooled_ref, w1_ref, b1_ref, w2_ref, b2_ref, out_ref):
    hh = jnp.maximum(jnp.dot(pooled_ref[...], w1_ref[...],
                             preferred_element_type=jnp.float32) + b1_ref[...], 0.0)
    out_ref[...] = jnp.dot(hh, w2_ref[...], preferred_element_type=jnp.float32) + b2_ref[...]


def _head(pooled, w1, b1, w2, b2):
    ncls = w2.shape[1]
    return pl.pallas_call(
        _head_body,
        out_shape=jax.ShapeDtypeStruct((pooled.shape[0], ncls), jnp.float32),
    )(pooled, w1, b1[None, :], w2, b2[None, :])


def _bn(x, g, bt):
    mean = jnp.mean(x, axis=0)
    var = jnp.var(x, axis=0)
    return g * (x - mean) * lax.rsqrt(var + EPS) + bt


def _upd(x, p):
    hh = jax.nn.relu(x @ p["W1"] + p["b1"])
    hh = jax.nn.relu(hh @ p["W2"] + p["b2"])
    return _bn(hh, p["g"], p["bt"])


def kernel(x0, x1, x2, up_index_0, up_attr_idx_0, up_index_1, up_attr_idx_1,
           down_index_1, down_attr_idx_1, down_index_2, down_attr_idx_2,
           batch0, batch1, batch2, params):
    N0, N1, N2 = x0.shape[0], x1.shape[0], x2.shape[0]
    E0, E1u = up_attr_idx_0.shape[0], up_attr_idx_1.shape[0]
    E1d, E2 = down_attr_idx_1.shape[0], down_attr_idx_2.shape[0]

    prep_u0 = _prep_edges(up_index_0[0], up_attr_idx_0, up_index_0[1], N0, N1, N0)
    prep_u1 = _prep_edges(up_index_1[0], up_attr_idx_1, up_index_1[1], N1, N2, N1)
    prep_d1 = _prep_edges(down_index_1[0], down_attr_idx_1, down_index_1[1], N1, N0, N1)
    prep_d2 = _prep_edges(down_index_2[0], down_attr_idx_2, down_index_2[1], N2, N1, N2)

    cnts = [None, None, None, None]
    xs = [x0, x1, x2]
    for lp in params["layers"]:
        x0_, x1_, x2_ = xs
        d = x0_.shape[1]
        Wu, bu = lp["up"]["W"], lp["up"]["b"]
        Wd, bd = lp["down"]["W"], lp["down"]["b"]
        agg0, cnts[0] = _adj_agg_sc(x0_ @ Wu[:d] + bu, x1_ @ Wu[d:], prep_u0,
                                    lp["up"], E0, N0, cnts[0])
        agg1u, cnts[1] = _adj_agg_sc(x1_ @ Wu[:d] + bu, x2_ @ Wu[d:], prep_u1,
                                     lp["up"], E1u, N1, cnts[1])
        agg1d, cnts[2] = _adj_agg_sc(x1_ @ Wd[:d] + bd, x0_ @ Wd[d:], prep_d1,
                                     lp["down"], E1d, N1, cnts[2])
        agg2, cnts[3] = _adj_agg_sc(x2_ @ Wd[:d] + bd, x1_ @ Wd[d:], prep_d2,
                                    lp["down"], E2, N2, cnts[3])
        xs = [_upd(x0_ + agg0, lp["upd"]), _upd(x1_ + agg1u + agg1d, lp["upd"]),
              _upd(x2_ + agg2, lp["upd"])]
    pooled = (jax.ops.segment_sum(xs[0], batch0, num_segments=B)
              + jax.ops.segment_sum(xs[1], batch1, num_segments=B)
              + jax.ops.segment_sum(xs[2], batch2, num_segments=B))
    return _head(pooled, params["lin1W"], params["lin1b"], params["lin2W"], params["lin2b"])
